# baseline (speedup 1.0000x reference)
import jax
import jax.numpy as jnp
from jax import lax
from jax.experimental import pallas as pl
from jax.experimental.pallas import tpu as pltpu
from jax.experimental.pallas import tpu_sc as plsc

NC, NS, L = 2, 16, 16
NW = NC * NS
B, LEN, V, D = 16384, 200, 10, 10
BAGS_PER_W = B // NW
GROUPS = BAGS_PER_W // L
XW = BAGS_PER_W * LEN
CSTRIDE = 17
CSLICE = L * CSTRIDE + L
WPAD = 128
WOFF = 8


def _sc_body(x_hbm, w_hbm, out_hbm, xbuf, wbuf, cnt, outbuf):
    wid = lax.axis_index("s") * NC + lax.axis_index("c")
    base = pl.multiple_of(wid * XW, 8)
    pltpu.sync_copy(x_hbm.at[pl.ds(base, XW)], xbuf)
    pltpu.sync_copy(w_hbm, wbuf)

    lane = lax.iota(jnp.int32, L)
    rowbase = lane * LEN
    lane17 = lane * CSTRIDE
    ones = jnp.ones((L,), jnp.int32)
    zeros = jnp.zeros((L,), jnp.int32)

    @plsc.parallel_loop(0, GROUPS)
    def group(g):
        gbase = g * (L * LEN)
        cbase = g * CSLICE
        for i in range(L):
            cnt[pl.ds(cbase + i * L, L)] = zeros
        cb = cbase + lane17
        xb = rowbase + gbase

        @plsc.parallel_loop(0, LEN, unroll=16)
        def step(l):
            idx = plsc.load_gather(xbuf, [xb + l])
            plsc.addupdate_scatter(cnt, [idx + cb], ones)

        accs = [jnp.zeros((L,), jnp.float32) for _ in range(D)]
        for v in range(V):
            cv = plsc.load_gather(cnt, [cb + v])
            cvf = cv.astype(jnp.float32) * (1.0 / LEN)
            for d in range(D):
                wv = plsc.load_gather(
                    wbuf, [jnp.full((L,), WOFF + v * D + d, jnp.int32)])
                accs[d] = accs[d] + cvf * wv
        outaddr = (g * L + lane) * D
        for d in range(D):
            plsc.store_scatter(outbuf, [outaddr + d], accs[d])

    pltpu.sync_copy(outbuf,
                    out_hbm.at[pl.ds(pl.multiple_of(wid * BAGS_PER_W * D, 8),
                                     BAGS_PER_W * D)])


_sc_call = pl.kernel(
    _sc_body,
    out_type=jax.ShapeDtypeStruct((B * D,), jnp.float32),
    mesh=plsc.VectorSubcoreMesh(core_axis_name="c", subcore_axis_name="s"),
    scratch_types=[
        pltpu.VMEM((XW,), jnp.int32),
        pltpu.VMEM((WPAD,), jnp.float32),
        pltpu.VMEM((GROUPS * CSLICE,), jnp.int32),
        pltpu.VMEM((BAGS_PER_W * D,), jnp.float32),
    ],
    compiler_params=pltpu.CompilerParams(needs_layout_passes=False),
)


def kernel(x, weight):
    xf = x.reshape(-1)
    wf = jnp.concatenate(
        [jnp.zeros((WOFF,), jnp.float32), weight.reshape(-1),
         jnp.zeros((WPAD - WOFF - V * D,), jnp.float32)])
    return _sc_call(xf, wf).reshape(B, D)


# speedup vs baseline: 1.0003x; 1.0003x over previous
import jax
import jax.numpy as jnp
from jax import lax
from jax.experimental import pallas as pl
from jax.experimental.pallas import tpu as pltpu
from jax.experimental.pallas import tpu_sc as plsc

NC, NS, L = 2, 16, 16
NW = NC * NS
B, LEN, V, D = 16384, 200, 10, 10
BAGS_PER_W = B // NW
GROUPS = BAGS_PER_W // L
XW = BAGS_PER_W * LEN
CSTRIDE = 17
CSLICE = L * CSTRIDE + L
WPAD = 128
WOFF = 8


def _sc_body(x_hbm, w_hbm, out_hbm, xbuf, wbuf, cnt, outbuf):
    wid = lax.axis_index("s") * NC + lax.axis_index("c")
    base = pl.multiple_of(wid * XW, 8)
    pltpu.sync_copy(x_hbm.at[pl.ds(base, XW)], xbuf)
    pltpu.sync_copy(w_hbm, wbuf)

    lane = lax.iota(jnp.int32, L)
    rowbase = lane * LEN
    lane17 = lane * CSTRIDE
    ones = jnp.ones((L,), jnp.int32)
    zeros = jnp.zeros((L,), jnp.int32)

    @plsc.parallel_loop(0, GROUPS)
    def group(g):
        gbase = g * (L * LEN)
        cbase = g * CSLICE
        for i in range(CSLICE // L):
            cnt[pl.ds(cbase + i * L, L)] = zeros
        cb = cbase + lane17
        xb = rowbase + gbase

        @plsc.parallel_loop(0, LEN, unroll=16)
        def step(l):
            idx = plsc.load_gather(xbuf, [xb + l])
            plsc.addupdate_scatter(cnt, [idx + cb], ones)

        accs = [jnp.zeros((L,), jnp.float32) for _ in range(D)]
        for v in range(V):
            cv = plsc.load_gather(cnt, [cb + v])
            cvf = cv.astype(jnp.float32) * (1.0 / LEN)
            for d in range(D):
                wv = plsc.load_gather(
                    wbuf, [jnp.full((L,), WOFF + v * D + d, jnp.int32)])
                accs[d] = accs[d] + cvf * wv
        outaddr = (g * L + lane) * D
        for d in range(D):
            plsc.store_scatter(outbuf, [outaddr + d], accs[d])

    pltpu.sync_copy(outbuf,
                    out_hbm.at[pl.ds(pl.multiple_of(wid * BAGS_PER_W * D, 8),
                                     BAGS_PER_W * D)])


_sc_call = pl.kernel(
    _sc_body,
    out_type=jax.ShapeDtypeStruct((B * D,), jnp.float32),
    mesh=plsc.VectorSubcoreMesh(core_axis_name="c", subcore_axis_name="s"),
    scratch_types=[
        pltpu.VMEM((XW,), jnp.int32),
        pltpu.VMEM((WPAD,), jnp.float32),
        pltpu.VMEM((GROUPS * CSLICE,), jnp.int32),
        pltpu.VMEM((BAGS_PER_W * D,), jnp.float32),
    ],
    compiler_params=pltpu.CompilerParams(needs_layout_passes=False),
)


def kernel(x, weight):
    xf = x.reshape(-1)
    wf = jnp.concatenate(
        [jnp.zeros((WOFF,), jnp.float32), weight.reshape(-1),
         jnp.zeros((WPAD - WOFF - V * D,), jnp.float32)])
    return _sc_call(xf, wf).reshape(B, D)


# speedup vs baseline: 1.0516x; 1.0513x over previous
import jax
import jax.numpy as jnp
from jax import lax
from jax.experimental import pallas as pl
from jax.experimental.pallas import tpu as pltpu
from jax.experimental.pallas import tpu_sc as plsc

NC, NS, L = 2, 16, 16
NW = NC * NS
B, LEN, V, D = 16384, 200, 10, 10
BAGS_PER_W = B // NW
GROUPS = BAGS_PER_W // L
XW = BAGS_PER_W * LEN
CSTRIDE = 17
CSLICE = L * CSTRIDE + L
KU = 20
WPAD = 128
WOFF = 8


def _sc_body(x_hbm, w_hbm, out_hbm, xbuf, wbuf, cnt, outbuf):
    wid = lax.axis_index("s") * NC + lax.axis_index("c")
    base = pl.multiple_of(wid * XW, 8)
    pltpu.sync_copy(x_hbm.at[pl.ds(base, XW)], xbuf)
    pltpu.sync_copy(w_hbm, wbuf)

    lane = lax.iota(jnp.int32, L)
    rowbase = lane * LEN
    lane17 = lane * CSTRIDE
    ones = jnp.ones((L,), jnp.int32)
    zeros = jnp.zeros((L,), jnp.int32)

    @plsc.parallel_loop(0, GROUPS)
    def group(g):
        gbase = g * (L * LEN)
        cbase = g * CSLICE
        for i in range(CSLICE // L):
            cnt[pl.ds(cbase + i * L, L)] = zeros
        cb = cbase + lane17
        xb = rowbase + gbase

        @plsc.parallel_loop(0, LEN, step=KU)
        def step(l):
            idxs = [plsc.load_gather(xbuf, [xb + (l + k)]) for k in range(KU)]
            for k in range(KU):
                plsc.addupdate_scatter(cnt, [idxs[k] + cb], ones)

        accs = [jnp.zeros((L,), jnp.float32) for _ in range(D)]
        for v in range(V):
            cv = plsc.load_gather(cnt, [cb + v])
            cvf = cv.astype(jnp.float32) * (1.0 / LEN)
            for d in range(D):
                wv = plsc.load_gather(
                    wbuf, [jnp.full((L,), WOFF + v * D + d, jnp.int32)])
                accs[d] = accs[d] + cvf * wv
        outaddr = (g * L + lane) * D
        for d in range(D):
            plsc.store_scatter(outbuf, [outaddr + d], accs[d])

    pltpu.sync_copy(outbuf,
                    out_hbm.at[pl.ds(pl.multiple_of(wid * BAGS_PER_W * D, 8),
                                     BAGS_PER_W * D)])


_sc_call = pl.kernel(
    _sc_body,
    out_type=jax.ShapeDtypeStruct((B * D,), jnp.float32),
    mesh=plsc.VectorSubcoreMesh(core_axis_name="c", subcore_axis_name="s"),
    scratch_types=[
        pltpu.VMEM((XW,), jnp.int32),
        pltpu.VMEM((WPAD,), jnp.float32),
        pltpu.VMEM((GROUPS * CSLICE,), jnp.int32),
        pltpu.VMEM((BAGS_PER_W * D,), jnp.float32),
    ],
    compiler_params=pltpu.CompilerParams(needs_layout_passes=False),
)


def kernel(x, weight):
    xf = x.reshape(-1)
    wf = jnp.concatenate(
        [jnp.zeros((WOFF,), jnp.float32), weight.reshape(-1),
         jnp.zeros((WPAD - WOFF - V * D,), jnp.float32)])
    return _sc_call(xf, wf).reshape(B, D)


# speedup vs baseline: 1.0529x; 1.0012x over previous
import jax
import jax.numpy as jnp
from jax import lax
from jax.experimental import pallas as pl
from jax.experimental.pallas import tpu as pltpu
from jax.experimental.pallas import tpu_sc as plsc

NC, NS, L = 2, 16, 16
NW = NC * NS
B, LEN, V, D = 16384, 200, 10, 10
BAGS_PER_W = B // NW
GROUPS = BAGS_PER_W // L
XW = BAGS_PER_W * LEN
CSTRIDE = 17
CSLICE = L * CSTRIDE + L
KU = 20
WPAD = 128
WOFF = 8


def _sc_body(x_hbm, w_hbm, out_hbm, xbuf, wbuf, cnt, outbuf):
    wid = lax.axis_index("s") * NC + lax.axis_index("c")
    base = pl.multiple_of(wid * XW, 8)
    pltpu.sync_copy(x_hbm.at[pl.ds(base, XW)], xbuf)
    pltpu.sync_copy(w_hbm, wbuf)

    lane = lax.iota(jnp.int32, L)
    rowbase = lane * LEN
    lane17 = lane * CSTRIDE
    ones = jnp.ones((L,), jnp.int32)
    zeros = jnp.zeros((L,), jnp.int32)

    @plsc.parallel_loop(0, GROUPS)
    def group(g):
        gbase = g * (L * LEN)
        cbase = g * CSLICE
        for i in range(CSLICE // L):
            cnt[pl.ds(cbase + i * L, L)] = zeros
        cb = cbase + lane17
        xb = rowbase + gbase
        xb3 = xb + 3 * lane
        thr = xb + LEN

        @plsc.parallel_loop(0, LEN, step=KU)
        def step(l):
            idxs = []
            for k in range(KU):
                p = xb3 + (l + k)
                p = jnp.where(p >= thr, p - LEN, p)
                idxs.append(plsc.load_gather(xbuf, [p]))
            for k in range(KU):
                plsc.addupdate_scatter(cnt, [idxs[k] + cb], ones)

        accs = [jnp.zeros((L,), jnp.float32) for _ in range(D)]
        for v in range(V):
            cv = plsc.load_gather(cnt, [cb + v])
            cvf = cv.astype(jnp.float32) * (1.0 / LEN)
            for d in range(D):
                wv = plsc.load_gather(
                    wbuf, [jnp.full((L,), WOFF + v * D + d, jnp.int32)])
                accs[d] = accs[d] + cvf * wv
        outaddr = (g * L + lane) * D
        for d in range(D):
            plsc.store_scatter(outbuf, [outaddr + d], accs[d])

    pltpu.sync_copy(outbuf,
                    out_hbm.at[pl.ds(pl.multiple_of(wid * BAGS_PER_W * D, 8),
                                     BAGS_PER_W * D)])


_sc_call = pl.kernel(
    _sc_body,
    out_type=jax.ShapeDtypeStruct((B * D,), jnp.float32),
    mesh=plsc.VectorSubcoreMesh(core_axis_name="c", subcore_axis_name="s"),
    scratch_types=[
        pltpu.VMEM((XW,), jnp.int32),
        pltpu.VMEM((WPAD,), jnp.float32),
        pltpu.VMEM((GROUPS * CSLICE,), jnp.int32),
        pltpu.VMEM((BAGS_PER_W * D,), jnp.float32),
    ],
    compiler_params=pltpu.CompilerParams(needs_layout_passes=False),
)


def kernel(x, weight):
    xf = x.reshape(-1)
    wf = jnp.concatenate(
        [jnp.zeros((WOFF,), jnp.float32), weight.reshape(-1),
         jnp.zeros((WPAD - WOFF - V * D,), jnp.float32)])
    return _sc_call(xf, wf).reshape(B, D)
